# baseline (speedup 1.0000x reference)
import functools

import jax
import jax.numpy as jnp
from jax import lax
from jax.experimental import pallas as pl
from jax.experimental.pallas import tpu as pltpu
from jax.experimental.pallas import tpu_sc as plsc

_N = 10000
_E = 320000
_H = 128
_G = 64

_NC = 2
_NS = 16
_NW = _NC * _NS
_EPW = _E // _NW
_CH = 80
_NCHUNK = _EPW // _CH
_RCH = 80
_NRCH = _N // _RCH
_RPT = -(-_NRCH // _NS)
_IR = 8
_ZR = 16
_NZCH = _N // _ZR
_ZPT = -(-_NZCH // _NS)


def _sc_agg(x, src3, dst3):
    mesh = plsc.VectorSubcoreMesh(core_axis_name="c", subcore_axis_name="s")

    @functools.partial(
        pl.kernel,
        out_type=jax.ShapeDtypeStruct((_NC, _N, _H), jnp.float32),
        mesh=mesh,
        scratch_types=[
            pltpu.VMEM((_CH, _H), jnp.float32),
            pltpu.VMEM((_CH, _H), jnp.float32),
            pltpu.VMEM((_CH, _H), jnp.float32),
            pltpu.VMEM((_CH, _H), jnp.float32),
            pltpu.VMEM((_IR, 2, _CH), jnp.int32),
            pltpu.VMEM((_ZR, _H), jnp.float32),
            pltpu.VMEM_SHARED((_N, _H), jnp.float32),
        ] + [pltpu.SemaphoreType.DMA] * (4 + _IR + _IR + 4 + 1),
    )
    def k(x_hbm, src_hbm, dst_hbm, out_hbm,
          rows0, rows1, rows2, rows3, idxr, zbuf, agg_sh, *sems):
        c = lax.axis_index("c")
        s = lax.axis_index("s")
        wid = s * _NC + c
        rows = (rows0, rows1, rows2, rows3)
        semg = sems[0:4]
        semis = sems[4:4 + _IR]
        semid = sems[4 + _IR:4 + 2 * _IR]
        semsc = sems[4 + 2 * _IR:8 + 2 * _IR]
        semz = sems[8 + 2 * _IR]

        def idx_start(j, ti):
            pltpu.make_async_copy(
                src_hbm.at[wid].at[j], idxr.at[ti, 0], semis[ti]).start()
            pltpu.make_async_copy(
                dst_hbm.at[wid].at[j], idxr.at[ti, 1], semid[ti]).start()

        def idx_wait(ti, which):
            sem = semis[ti] if which == 0 else semid[ti]
            pltpu.make_async_copy(
                src_hbm.at[wid].at[0], idxr.at[ti, which], sem).wait()

        def gather_start(tr, ti):
            pltpu.make_async_copy(
                x_hbm.at[idxr.at[ti, 0]], rows[tr], semg[tr]).start()

        def gather_wait(tr, ti):
            pltpu.make_async_copy(
                x_hbm.at[idxr.at[ti, 0]], rows[tr], semg[tr]).wait()

        def scatter_start(tr, ti):
            pltpu.async_copy(
                rows[tr], agg_sh.at[idxr.at[ti, 1]], semsc[tr], add=True)

        def scatter_wait(tr, ti):
            pltpu.make_async_copy(
                rows[tr], agg_sh.at[idxr.at[ti, 1]], semsc[tr]).wait()

        for j0 in range(6):
            idx_start(j0, j0)
        for j0 in range(3):
            idx_wait(j0, 0)
            gather_start(j0 % 4, j0)

        @pl.loop(0, _ZR)
        def _(r):
            for j in range(_H // 16):
                zbuf[r, pl.ds(j * 16, 16)] = jnp.zeros((16,), jnp.float32)

        @pl.loop(0, _ZPT)
        def _(t):
            cid = t * _NS + s

            @pl.when(cid < _NZCH)
            def _():
                off = pl.multiple_of(cid * _ZR, 8)
                pltpu.make_async_copy(
                    zbuf, agg_sh.at[pl.ds(off, _ZR)], semz).start()

        @pl.loop(0, _ZPT)
        def _(t):
            cid = t * _NS + s

            @pl.when(cid < _NZCH)
            def _():
                pltpu.make_async_copy(
                    zbuf, agg_sh.at[pl.ds(0, _ZR)], semz).wait()

        plsc.subcore_barrier()

        @pl.loop(0, -(-_NCHUNK // _IR))
        def _(m):
            for t in range(_IR):
                j = m * _IR + t

                @pl.when(j < _NCHUNK)
                def _():
                    gather_wait(t % 4, t % _IR)
                    idx_wait(t % _IR, 1)
                    scatter_start(t % 4, t % _IR)

                @pl.when(j + 6 < _NCHUNK)
                def _():
                    idx_start(j + 6, (t + 6) % _IR)

                @pl.when(j + 3 < _NCHUNK)
                def _():
                    @pl.when(j >= 1)
                    def _():
                        scatter_wait((t + 3) % 4, (t + 7) % _IR)

                    idx_wait((t + 3) % _IR, 0)
                    gather_start((t + 3) % 4, (t + 3) % _IR)

        for cc in range(_NCHUNK - 4, _NCHUNK):
            scatter_wait(cc % 4, cc % _IR)

        plsc.subcore_barrier()

        @pl.loop(0, _RPT)
        def _(t):
            cid = t * _NS + s

            @pl.when(cid < _NRCH)
            def _():
                off = pl.multiple_of(cid * _RCH, 8)
                pltpu.make_async_copy(
                    agg_sh.at[pl.ds(off, _RCH)],
                    out_hbm.at[c].at[pl.ds(off, _RCH)], semz).start()

        @pl.loop(0, _RPT)
        def _(t):
            cid = t * _NS + s

            @pl.when(cid < _NRCH)
            def _():
                pltpu.make_async_copy(
                    agg_sh.at[pl.ds(0, _RCH)],
                    out_hbm.at[c].at[pl.ds(0, _RCH)], semz).wait()

    return k(x, src3, dst3)


def _tc_layer(x, aggp, Wa, ba, Wb, bb, gamma, beta):
    def body(x_ref, a_ref, wa_ref, ba_ref, wb_ref, bb_ref, g_ref, be_ref, o_ref):
        h = x_ref[...] + a_ref[0] + a_ref[1]
        z = jnp.dot(h, wa_ref[...], preferred_element_type=jnp.float32)
        z = jnp.maximum(z + ba_ref[...], 0.0)
        z = jnp.dot(z, wb_ref[...], preferred_element_type=jnp.float32)
        z = jnp.maximum(z + bb_ref[...], 0.0)
        m = jnp.mean(z, axis=0, keepdims=True)
        d = z - m
        v = jnp.mean(d * d, axis=0, keepdims=True)
        o_ref[...] = d / jnp.sqrt(v + 1e-5) * g_ref[...] + be_ref[...]

    return pl.pallas_call(
        body, out_shape=jax.ShapeDtypeStruct((_N, _H), jnp.float32),
    )(x, aggp, Wa, ba.reshape(1, _H), Wb, bb.reshape(1, _H),
      gamma.reshape(1, _H), beta.reshape(1, _H))


def _tc_pool(x1, x2, x3, batch2d, W1, W2, W3, b):
    def body(x1_ref, x2_ref, x3_ref, bt_ref, w1_ref, w2_ref, w3_ref, b_ref,
             o_ref):
        gid = lax.broadcasted_iota(jnp.int32, (_G, _N), 0)
        onehot = (gid == bt_ref[...]).astype(jnp.float32)
        counts = jnp.sum(onehot, axis=1, keepdims=True)
        pt = onehot / jnp.maximum(counts, 1.0)
        acc = jnp.dot(jnp.dot(pt, x1_ref[...], preferred_element_type=jnp.float32, precision=lax.Precision.HIGHEST),
                      w1_ref[...], preferred_element_type=jnp.float32)
        acc += jnp.dot(jnp.dot(pt, x2_ref[...], preferred_element_type=jnp.float32, precision=lax.Precision.HIGHEST),
                       w2_ref[...], preferred_element_type=jnp.float32)
        acc += jnp.dot(jnp.dot(pt, x3_ref[...], preferred_element_type=jnp.float32, precision=lax.Precision.HIGHEST),
                       w3_ref[...], preferred_element_type=jnp.float32)
        o_ref[...] = acc + b_ref[...]

    return pl.pallas_call(
        body, out_shape=jax.ShapeDtypeStruct((_G, _H), jnp.float32),
    )(x1, x2, x3, batch2d, W1, W2, W3, b)


def kernel(x, edge_index, batch, params):
    src3 = edge_index[0].reshape(_NW, _NCHUNK, _CH)
    dst3 = edge_index[1].reshape(_NW, _NCHUNK, _CH)
    h = x
    feats = []
    for l in range(3):
        aggp = _sc_agg(h, src3, dst3)
        h = _tc_layer(h, aggp,
                      params['l%d_Wa' % l], params['l%d_ba' % l],
                      params['l%d_Wb' % l], params['l%d_bb' % l],
                      params['l%d_gamma' % l], params['l%d_beta' % l])
        feats.append(h)
    W = params['lin_W']
    return _tc_pool(feats[0], feats[1], feats[2], batch.reshape(1, _N),
                    W[0:_H], W[_H:2 * _H], W[2 * _H:3 * _H],
                    params['lin_b'].reshape(1, _H))


# speedup vs baseline: 1.0711x; 1.0711x over previous
import functools

import jax
import jax.numpy as jnp
from jax import lax
from jax.experimental import pallas as pl
from jax.experimental.pallas import tpu as pltpu
from jax.experimental.pallas import tpu_sc as plsc

_N = 10000
_E = 320000
_H = 128
_G = 64

_NC = 2
_NS = 16
_NW = _NC * _NS
_EPW = _E // _NW
_CH = 80
_NCHUNK = _EPW // _CH
_RCH = 80
_NRCH = _N // _RCH
_RPT = -(-_NRCH // _NS)
_IR = 8
_ZR = 16
_NZCH = _N // _ZR
_ZPT = -(-_NZCH // _NS)


def _sc_agg(x, src3, dst3):
    mesh = plsc.VectorSubcoreMesh(core_axis_name="c", subcore_axis_name="s")

    @functools.partial(
        pl.kernel,
        out_type=jax.ShapeDtypeStruct((_NC, _N, _H), jnp.float32),
        mesh=mesh,
        scratch_types=[
            pltpu.VMEM((_CH, _H), jnp.float32),
            pltpu.VMEM((_CH, _H), jnp.float32),
            pltpu.VMEM((_CH, _H), jnp.float32),
            pltpu.VMEM((_CH, _H), jnp.float32),
            pltpu.VMEM((_IR, 2, _CH), jnp.int32),
            pltpu.VMEM((_ZR, _H), jnp.float32),
            pltpu.VMEM_SHARED((_N, _H), jnp.float32),
        ] + [pltpu.SemaphoreType.DMA] * (4 + _IR + _IR + 1),
    )
    def k(x_hbm, src_hbm, dst_hbm, out_hbm,
          rows0, rows1, rows2, rows3, idxr, zbuf, agg_sh, *sems):
        c = lax.axis_index("c")
        s = lax.axis_index("s")
        wid = s * _NC + c
        rows = (rows0, rows1, rows2, rows3)
        semg = sems[0:4]
        semis = sems[4:4 + _IR]
        semid = sems[4 + _IR:4 + 2 * _IR]
        semz = sems[4 + 2 * _IR]

        def idx_start(j, ti):
            pltpu.make_async_copy(
                src_hbm.at[wid].at[j], idxr.at[ti, 0], semis[ti]).start()
            pltpu.make_async_copy(
                dst_hbm.at[wid].at[j], idxr.at[ti, 1], semid[ti]).start()

        def idx_wait(ti, which):
            sem = semis[ti] if which == 0 else semid[ti]
            pltpu.make_async_copy(
                src_hbm.at[wid].at[0], idxr.at[ti, which], sem).wait()

        def gather_start(tr, ti):
            pltpu.make_async_copy(
                x_hbm.at[idxr.at[ti, 0]], rows[tr], semg[tr]).start()

        def gather_wait(tr, ti):
            pltpu.make_async_copy(
                x_hbm.at[idxr.at[ti, 0]], rows[tr], semg[tr]).wait()

        for j0 in range(6):
            idx_start(j0, j0)
        for j0 in range(3):
            idx_wait(j0, 0)
            gather_start(j0 % 4, j0)

        @pl.loop(0, _ZR)
        def _(r):
            for j in range(_H // 16):
                zbuf[r, pl.ds(j * 16, 16)] = jnp.zeros((16,), jnp.float32)

        @pl.loop(0, _ZPT)
        def _(t):
            cid = t * _NS + s

            @pl.when(cid < _NZCH)
            def _():
                off = pl.multiple_of(cid * _ZR, 8)
                pltpu.make_async_copy(
                    zbuf, agg_sh.at[pl.ds(off, _ZR)], semz).start()

        @pl.loop(0, _ZPT)
        def _(t):
            cid = t * _NS + s

            @pl.when(cid < _NZCH)
            def _():
                pltpu.make_async_copy(
                    zbuf, agg_sh.at[pl.ds(0, _ZR)], semz).wait()

        plsc.subcore_barrier()

        @pl.loop(0, -(-_NCHUNK // _IR))
        def _(m):
            for t in range(_IR):
                j = m * _IR + t

                @pl.when(j < _NCHUNK)
                def _():
                    gather_wait(t % 4, t % _IR)
                    idx_wait(t % _IR, 1)
                    pltpu.sync_copy(rows[t % 4], agg_sh.at[idxr.at[t % _IR, 1]],
                                    add=True)

                @pl.when(j + 6 < _NCHUNK)
                def _():
                    idx_start(j + 6, (t + 6) % _IR)

                @pl.when(j + 3 < _NCHUNK)
                def _():
                    idx_wait((t + 3) % _IR, 0)
                    gather_start((t + 3) % 4, (t + 3) % _IR)

        plsc.subcore_barrier()

        @pl.loop(0, _RPT)
        def _(t):
            cid = t * _NS + s

            @pl.when(cid < _NRCH)
            def _():
                off = pl.multiple_of(cid * _RCH, 8)
                pltpu.make_async_copy(
                    agg_sh.at[pl.ds(off, _RCH)],
                    out_hbm.at[c].at[pl.ds(off, _RCH)], semz).start()

        @pl.loop(0, _RPT)
        def _(t):
            cid = t * _NS + s

            @pl.when(cid < _NRCH)
            def _():
                pltpu.make_async_copy(
                    agg_sh.at[pl.ds(0, _RCH)],
                    out_hbm.at[c].at[pl.ds(0, _RCH)], semz).wait()

    return k(x, src3, dst3)


def _pooler(bt, G, N):
    gid = lax.broadcasted_iota(jnp.int32, (G, N), 0)
    onehot = (gid == bt).astype(jnp.float32)
    counts = jnp.sum(onehot, axis=1, keepdims=True)
    return onehot / jnp.maximum(counts, 1.0)


def _gin_dense(x_ref, a_ref, wa_ref, ba_ref, wb_ref, bb_ref, g_ref, be_ref):
    h = x_ref[...] + a_ref[0] + a_ref[1]
    z = jnp.dot(h, wa_ref[...], preferred_element_type=jnp.float32)
    z = jnp.maximum(z + ba_ref[...], 0.0)
    z = jnp.dot(z, wb_ref[...], preferred_element_type=jnp.float32)
    z = jnp.maximum(z + bb_ref[...], 0.0)
    m = jnp.mean(z, axis=0, keepdims=True)
    d = z - m
    v = jnp.mean(d * d, axis=0, keepdims=True)
    return d / jnp.sqrt(v + 1e-5) * g_ref[...] + be_ref[...]


def _tc_layer(x, aggp, Wa, ba, Wb, bb, gamma, beta, batch2d):
    def body(x_ref, a_ref, wa_ref, ba_ref, wb_ref, bb_ref, g_ref, be_ref,
             bt_ref, o_ref, p_ref):
        zb = _gin_dense(x_ref, a_ref, wa_ref, ba_ref, wb_ref, bb_ref,
                        g_ref, be_ref)
        o_ref[...] = zb
        pt = _pooler(bt_ref[...], _G, _N)
        p_ref[...] = jnp.dot(pt, zb, preferred_element_type=jnp.float32,
                             precision=lax.Precision.HIGHEST)

    return pl.pallas_call(
        body, out_shape=[jax.ShapeDtypeStruct((_N, _H), jnp.float32),
                         jax.ShapeDtypeStruct((_G, _H), jnp.float32)],
    )(x, aggp, Wa, ba.reshape(1, _H), Wb, bb.reshape(1, _H),
      gamma.reshape(1, _H), beta.reshape(1, _H), batch2d)


def _tc_layer_final(x, aggp, Wa, ba, Wb, bb, gamma, beta, batch2d,
                    pool1, pool2, W1, W2, W3, b):
    def body(x_ref, a_ref, wa_ref, ba_ref, wb_ref, bb_ref, g_ref, be_ref,
             bt_ref, p1_ref, p2_ref, w1_ref, w2_ref, w3_ref, b_ref, o_ref):
        zb = _gin_dense(x_ref, a_ref, wa_ref, ba_ref, wb_ref, bb_ref,
                        g_ref, be_ref)
        pt = _pooler(bt_ref[...], _G, _N)
        p3 = jnp.dot(pt, zb, preferred_element_type=jnp.float32,
                     precision=lax.Precision.HIGHEST)
        acc = jnp.dot(p1_ref[...], w1_ref[...],
                      preferred_element_type=jnp.float32)
        acc += jnp.dot(p2_ref[...], w2_ref[...],
                       preferred_element_type=jnp.float32)
        acc += jnp.dot(p3, w3_ref[...], preferred_element_type=jnp.float32)
        o_ref[...] = acc + b_ref[...]

    return pl.pallas_call(
        body, out_shape=jax.ShapeDtypeStruct((_G, _H), jnp.float32),
    )(x, aggp, Wa, ba.reshape(1, _H), Wb, bb.reshape(1, _H),
      gamma.reshape(1, _H), beta.reshape(1, _H), batch2d,
      pool1, pool2, W1, W2, W3, b)


def kernel(x, edge_index, batch, params):
    src3 = edge_index[0].reshape(_NW, _NCHUNK, _CH)
    dst3 = edge_index[1].reshape(_NW, _NCHUNK, _CH)
    b2 = batch.reshape(1, _N)
    W = params['lin_W']
    h = x
    pools = []
    for l in range(2):
        aggp = _sc_agg(h, src3, dst3)
        h, p = _tc_layer(h, aggp,
                         params['l%d_Wa' % l], params['l%d_ba' % l],
                         params['l%d_Wb' % l], params['l%d_bb' % l],
                         params['l%d_gamma' % l], params['l%d_beta' % l], b2)
        pools.append(p)
    aggp = _sc_agg(h, src3, dst3)
    return _tc_layer_final(h, aggp,
                           params['l2_Wa'], params['l2_ba'],
                           params['l2_Wb'], params['l2_bb'],
                           params['l2_gamma'], params['l2_beta'], b2,
                           pools[0], pools[1],
                           W[0:_H], W[_H:2 * _H], W[2 * _H:3 * _H],
                           params['lin_b'].reshape(1, _H))
